# baseline (speedup 1.0000x reference)
import jax
import jax.numpy as jnp
from jax.experimental import pallas as pl
from jax.experimental.pallas import tpu as pltpu

_BLOCK_T = 1024


def _router_body(x_ref, w_ref, b_ref, o_ref):
    o_ref[...] = (
        jnp.dot(x_ref[...], w_ref[...], preferred_element_type=jnp.float32)
        + b_ref[...]
    )


def kernel(x, W, b):
    n_tokens, d_model = x.shape
    n_experts = W.shape[1]
    b2 = b.reshape(1, n_experts)
    return pl.pallas_call(
        _router_body,
        grid=(n_tokens // _BLOCK_T,),
        in_specs=[
            pl.BlockSpec((_BLOCK_T, d_model), lambda i: (i, 0)),
            pl.BlockSpec((d_model, n_experts), lambda i: (0, 0)),
            pl.BlockSpec((1, n_experts), lambda i: (0, 0)),
        ],
        out_specs=pl.BlockSpec((_BLOCK_T, n_experts), lambda i: (i, 0)),
        out_shape=jax.ShapeDtypeStruct((n_tokens, n_experts), jnp.float32),
        compiler_params=pltpu.CompilerParams(
            dimension_semantics=("arbitrary",),
            skip_device_barrier=True,
            disable_bounds_checks=True,
        ),
    )(x, W, b2)


# speedup vs baseline: 1.0038x; 1.0038x over previous
import jax
import jax.numpy as jnp
from jax.experimental import pallas as pl
from jax.experimental.pallas import tpu as pltpu

_BLOCK_T = 1024


def _router_body(x_ref, w_ref, b_ref, o_ref):
    o_ref[...] = (
        jnp.dot(x_ref[...], w_ref[...], preferred_element_type=jnp.float32)
        + b_ref[...][None, :]
    )


def kernel(x, W, b):
    n_tokens, d_model = x.shape
    n_experts = W.shape[1]
    return pl.pallas_call(
        _router_body,
        grid=(n_tokens // _BLOCK_T,),
        in_specs=[
            pl.BlockSpec((_BLOCK_T, d_model), lambda i: (i, 0)),
            pl.BlockSpec((d_model, n_experts), lambda i: (0, 0)),
            pl.BlockSpec((n_experts,), lambda i: (0,)),
        ],
        out_specs=pl.BlockSpec((_BLOCK_T, n_experts), lambda i: (i, 0)),
        out_shape=jax.ShapeDtypeStruct((n_tokens, n_experts), jnp.float32),
        compiler_params=pltpu.CompilerParams(
            dimension_semantics=("arbitrary",),
        ),
    )(x, W, b)
